# Initial kernel scaffold; baseline (speedup 1.0000x reference)
import functools

import jax
import jax.numpy as jnp
from jax import lax
from jax.experimental import pallas as pl
from jax.experimental.pallas import tpu as pltpu
from jax.experimental.pallas import tpu_sc as plsc

_LAM = 0.75
_GAMMA = 1.5
_NEG_WEIGHT = 300.0
_L2_REG_WEIGHT = 1e-4

_NC = 2
_NS = 16
_NW = _NC * _NS
_CH = 128


def _sc_gather(emb_table, beta_uD, beta_iD, nbr_flat, sc_flat,
               src, pos, neg_t, ii_flat_idx):
    b = src.shape[0]
    nneg_total = neg_t.shape[0]
    nii_total = ii_flat_idx.shape[0]
    d = emb_table.shape[1]

    b_w = b // _NW
    neg_w = nneg_total // _NW
    ii_w = nii_total // _NW
    assert b_w == _CH and neg_w % _CH == 0 and ii_w % _CH == 0

    mesh = plsc.VectorSubcoreMesh(core_axis_name="c", subcore_axis_name="s")
    f32 = jnp.float32

    @functools.partial(
        pl.kernel,
        out_type=[
            jax.ShapeDtypeStruct((b, d), f32),
            jax.ShapeDtypeStruct((b, d), f32),
            jax.ShapeDtypeStruct((nneg_total, d), f32),
            jax.ShapeDtypeStruct((nii_total, d), f32),
            jax.ShapeDtypeStruct((nii_total,), f32),
            jax.ShapeDtypeStruct((b,), f32),
            jax.ShapeDtypeStruct((b,), f32),
            jax.ShapeDtypeStruct((nneg_total,), f32),
        ],
        mesh=mesh,
        scratch_types=[
            pltpu.VMEM((_CH,), jnp.int32),
            pltpu.VMEM((_CH, 128), f32),
            pltpu.VMEM((_CH,), f32),
            pltpu.VMEM((ii_w,), jnp.int32),
        ],
    )
    def gather_kernel(emb_h, bu_h, bi_h, nbr_h, scf_h, src_h, pos_h, negt_h,
                      iidx_h, osrc_h, opos_h, oneg_h, oii_h, oiisc_h,
                      obu_h, obip_h, obin_h,
                      idx_v, rows_v, val_v, nbr_v):
        wid = lax.axis_index("s") * _NC + lax.axis_index("c")

        base = wid * b_w
        pltpu.sync_copy(src_h.at[pl.ds(base, _CH)], idx_v)
        pltpu.sync_copy(emb_h.at[idx_v], rows_v)
        pltpu.sync_copy(rows_v, osrc_h.at[pl.ds(base, _CH)])
        pltpu.sync_copy(bu_h.at[idx_v], val_v)
        pltpu.sync_copy(val_v, obu_h.at[pl.ds(base, _CH)])

        pltpu.sync_copy(pos_h.at[pl.ds(base, _CH)], idx_v)
        pltpu.sync_copy(emb_h.at[idx_v], rows_v)
        pltpu.sync_copy(rows_v, opos_h.at[pl.ds(base, _CH)])
        pltpu.sync_copy(bi_h.at[idx_v], val_v)
        pltpu.sync_copy(val_v, obip_h.at[pl.ds(base, _CH)])

        @pl.loop(0, neg_w // _CH)
        def _neg(c):
            nbase = wid * neg_w + c * _CH
            pltpu.sync_copy(negt_h.at[pl.ds(nbase, _CH)], idx_v)
            pltpu.sync_copy(emb_h.at[idx_v], rows_v)
            pltpu.sync_copy(rows_v, oneg_h.at[pl.ds(nbase, _CH)])
            pltpu.sync_copy(bi_h.at[idx_v], val_v)
            pltpu.sync_copy(val_v, obin_h.at[pl.ds(nbase, _CH)])

        @pl.loop(0, ii_w // _CH)
        def _iia(c):
            ibase = wid * ii_w + c * _CH
            pltpu.sync_copy(iidx_h.at[pl.ds(ibase, _CH)], idx_v)
            pltpu.sync_copy(nbr_h.at[idx_v], nbr_v.at[pl.ds(c * _CH, _CH)])
            pltpu.sync_copy(scf_h.at[idx_v], val_v)
            pltpu.sync_copy(val_v, oiisc_h.at[pl.ds(ibase, _CH)])

        @pl.loop(0, ii_w // _CH)
        def _iib(c):
            ibase = wid * ii_w + c * _CH
            pltpu.sync_copy(emb_h.at[nbr_v.at[pl.ds(c * _CH, _CH)]], rows_v)
            pltpu.sync_copy(rows_v, oii_h.at[pl.ds(ibase, _CH)])

    return gather_kernel(emb_table, beta_uD, beta_iD, nbr_flat, sc_flat,
                         src, pos, neg_t, ii_flat_idx)


def _loss_block(src_ref, pos_ref, neg_ref, ii_ref, bu_ref, bip_ref,
                bin_ref, iisc_ref, out_ref):
    b = pl.program_id(0)

    src = src_ref[...]
    pos = pos_ref[...]
    neg = neg_ref[...]
    ii = ii_ref[...]

    pos_score = jnp.sum(src * pos, axis=-1)
    neg_score = jnp.sum(src * neg, axis=-1)
    ii_score = jnp.sum(src * ii, axis=-1)

    bu = bu_ref[...]
    bip = bip_ref[...]
    bin_ = bin_ref[...]
    iisc = iisc_ref[...]

    def bce(x, target):
        return (jnp.maximum(x, 0.0) - x * target
                + jnp.log1p(jnp.exp(-jnp.abs(x))))

    pos_coe = 1.0 + _LAM * bu * bip
    neg_coe = 1.0 + _LAM * bu * bin_
    s_pos = jnp.sum(bce(pos_score, 1.0) * pos_coe)
    s_neg = jnp.sum(bce(neg_score, 0.0) * neg_coe)

    log_sig = jnp.minimum(ii_score, 0.0) - jnp.log1p(jnp.exp(-jnp.abs(ii_score)))
    s_i = jnp.sum(iisc * log_sig)

    s_l2 = (jnp.sum(src * src) + jnp.sum(pos * pos)
            + jnp.sum(neg * neg) + jnp.sum(ii * ii))

    n_pos = src_ref.shape[1] * pl.num_programs(0)
    n_neg = neg_ref.shape[0] * n_pos
    contrib = (s_pos / n_pos
               + (_NEG_WEIGHT / n_neg) * s_neg
               - _GAMMA * s_i
               + (0.5 * _L2_REG_WEIGHT) * s_l2)

    @pl.when(b == 0)
    def _():
        out_ref[0, 0] = 0.0
    out_ref[0, 0] += contrib


def _tc_loss(src_rows, pos_rows, neg_rows, ii_rows, ii_sc, bu, bip, bin_):
    b, d = src_rows.shape
    nneg = neg_rows.shape[0] // b
    topk = ii_rows.shape[0] // b
    bb = 128
    nblk = b // bb

    src3 = src_rows.reshape(1, b, d)
    pos3 = pos_rows.reshape(1, b, d)
    neg3 = neg_rows.reshape(nneg, b, d)
    ii3 = ii_rows.reshape(topk, b, d)
    bu2 = bu.reshape(1, b)
    bip2 = bip.reshape(1, b)
    bin2 = bin_.reshape(nneg, b)
    iisc2 = ii_sc.reshape(topk, b)

    out = pl.pallas_call(
        _loss_block,
        grid=(nblk,),
        in_specs=[
            pl.BlockSpec((1, bb, d), lambda i: (0, i, 0)),
            pl.BlockSpec((1, bb, d), lambda i: (0, i, 0)),
            pl.BlockSpec((nneg, bb, d), lambda i: (0, i, 0)),
            pl.BlockSpec((topk, bb, d), lambda i: (0, i, 0)),
            pl.BlockSpec((1, bb), lambda i: (0, i)),
            pl.BlockSpec((1, bb), lambda i: (0, i)),
            pl.BlockSpec((nneg, bb), lambda i: (0, i)),
            pl.BlockSpec((topk, bb), lambda i: (0, i)),
        ],
        out_specs=pl.BlockSpec((1, 1), lambda i: (0, 0)),
        out_shape=jax.ShapeDtypeStruct((1, 1), jnp.float32),
    )(src3, pos3, neg3, ii3, bu2, bip2, bin2, iisc2)
    return out[0, 0]


def kernel(emb_table, beta_uD, beta_iD, ii_topk_similarity_scores,
           src, pos, neg, ii_topk_neighbors):
    b, nneg = neg.shape
    topk = ii_topk_neighbors.shape[1]

    neg_t = neg.T.reshape(-1).astype(jnp.int32)
    ii_flat_idx = (pos[None, :] * topk
                   + jnp.arange(topk, dtype=pos.dtype)[:, None]
                   ).reshape(-1).astype(jnp.int32)
    nbr_flat = ii_topk_neighbors.reshape(-1).astype(jnp.int32)
    sc_flat = ii_topk_similarity_scores.reshape(-1)

    (src_rows, pos_rows, neg_rows, ii_rows, ii_sc, bu, bip, bin_) = _sc_gather(
        emb_table, beta_uD, beta_iD, nbr_flat, sc_flat,
        src.astype(jnp.int32), pos.astype(jnp.int32), neg_t, ii_flat_idx)

    return _tc_loss(src_rows, pos_rows, neg_rows, ii_rows, ii_sc, bu, bip, bin_)


# speedup vs baseline: 5.0302x; 5.0302x over previous
import functools

import jax
import jax.numpy as jnp
from jax import lax
from jax.experimental import pallas as pl
from jax.experimental.pallas import tpu as pltpu
from jax.experimental.pallas import tpu_sc as plsc

_LAM = 0.75
_GAMMA = 1.5
_NEG_WEIGHT = 300.0
_L2_REG_WEIGHT = 1e-4

_NC = 2
_NS = 16
_NW = _NC * _NS
_CH = 128


def _sc_gather(emb_table, beta_uD, beta_iD, nbr_flat, sc_flat,
               src, pos, neg_t, ii_flat_idx):
    b = src.shape[0]
    nneg_total = neg_t.shape[0]
    nii_total = ii_flat_idx.shape[0]
    d = emb_table.shape[1]

    b_w = b // _NW
    neg_w = nneg_total // _NW
    ii_w = nii_total // _NW
    assert b_w == _CH and neg_w % _CH == 0 and ii_w % _CH == 0

    mesh = plsc.VectorSubcoreMesh(core_axis_name="c", subcore_axis_name="s")
    f32 = jnp.float32

    @functools.partial(
        pl.kernel,
        out_type=[
            jax.ShapeDtypeStruct((b, d), f32),
            jax.ShapeDtypeStruct((b, d), f32),
            jax.ShapeDtypeStruct((nneg_total, d), f32),
            jax.ShapeDtypeStruct((nii_total, d), f32),
            jax.ShapeDtypeStruct((nii_total,), f32),
            jax.ShapeDtypeStruct((b,), f32),
            jax.ShapeDtypeStruct((b,), f32),
            jax.ShapeDtypeStruct((nneg_total,), f32),
        ],
        mesh=mesh,
        scratch_types=[
            pltpu.VMEM((_CH,), jnp.int32),
            pltpu.VMEM((_CH, 128), f32),
            pltpu.VMEM((_CH,), f32),
            pltpu.VMEM((ii_w,), jnp.int32),
        ],
    )
    def gather_kernel(emb_h, bu_h, bi_h, nbr_h, scf_h, src_h, pos_h, negt_h,
                      iidx_h, osrc_h, opos_h, oneg_h, oii_h, oiisc_h,
                      obu_h, obip_h, obin_h,
                      idx_v, rows_v, val_v, nbr_v):
        wid = lax.axis_index("s") * _NC + lax.axis_index("c")

        base = wid * b_w
        pltpu.sync_copy(src_h.at[pl.ds(base, _CH)], idx_v)
        pltpu.sync_copy(emb_h.at[idx_v], rows_v)
        pltpu.sync_copy(rows_v, osrc_h.at[pl.ds(base, _CH)])
        pltpu.sync_copy(bu_h.at[idx_v], val_v)
        pltpu.sync_copy(val_v, obu_h.at[pl.ds(base, _CH)])

        pltpu.sync_copy(pos_h.at[pl.ds(base, _CH)], idx_v)
        pltpu.sync_copy(emb_h.at[idx_v], rows_v)
        pltpu.sync_copy(rows_v, opos_h.at[pl.ds(base, _CH)])
        pltpu.sync_copy(bi_h.at[idx_v], val_v)
        pltpu.sync_copy(val_v, obip_h.at[pl.ds(base, _CH)])

        @pl.loop(0, neg_w // _CH)
        def _neg(c):
            nbase = wid * neg_w + c * _CH
            pltpu.sync_copy(negt_h.at[pl.ds(nbase, _CH)], idx_v)
            pltpu.sync_copy(emb_h.at[idx_v], rows_v)
            pltpu.sync_copy(rows_v, oneg_h.at[pl.ds(nbase, _CH)])
            pltpu.sync_copy(bi_h.at[idx_v], val_v)
            pltpu.sync_copy(val_v, obin_h.at[pl.ds(nbase, _CH)])

        @pl.loop(0, ii_w // _CH)
        def _iia(c):
            ibase = wid * ii_w + c * _CH
            pltpu.sync_copy(iidx_h.at[pl.ds(ibase, _CH)], idx_v)
            pltpu.sync_copy(nbr_h.at[idx_v], nbr_v.at[pl.ds(c * _CH, _CH)])
            pltpu.sync_copy(scf_h.at[idx_v], val_v)
            pltpu.sync_copy(val_v, oiisc_h.at[pl.ds(ibase, _CH)])

        @pl.loop(0, ii_w // _CH)
        def _iib(c):
            ibase = wid * ii_w + c * _CH
            pltpu.sync_copy(emb_h.at[nbr_v.at[pl.ds(c * _CH, _CH)]], rows_v)
            pltpu.sync_copy(rows_v, oii_h.at[pl.ds(ibase, _CH)])

    return gather_kernel(emb_table, beta_uD, beta_iD, nbr_flat, sc_flat,
                         src, pos, neg_t, ii_flat_idx)


def _loss_block(src_ref, pos_ref, neg_ref, ii_ref, bu_ref, bip_ref,
                bin_ref, iisc_ref, out_ref):
    b = pl.program_id(0)

    src = src_ref[...]
    pos = pos_ref[...]
    neg = neg_ref[...]
    ii = ii_ref[...]

    pos_score = jnp.sum(src * pos, axis=-1)
    neg_score = jnp.sum(src * neg, axis=-1)
    ii_score = jnp.sum(src * ii, axis=-1)

    bu = bu_ref[...]
    bip = bip_ref[...]
    bin_ = bin_ref[...]
    iisc = iisc_ref[...]

    def bce(x, target):
        return (jnp.maximum(x, 0.0) - x * target
                + jnp.log1p(jnp.exp(-jnp.abs(x))))

    pos_coe = 1.0 + _LAM * bu * bip
    neg_coe = 1.0 + _LAM * bu * bin_
    s_pos = jnp.sum(bce(pos_score, 1.0) * pos_coe)
    s_neg = jnp.sum(bce(neg_score, 0.0) * neg_coe)

    log_sig = jnp.minimum(ii_score, 0.0) - jnp.log1p(jnp.exp(-jnp.abs(ii_score)))
    s_i = jnp.sum(iisc * log_sig)

    s_l2 = (jnp.sum(src * src) + jnp.sum(pos * pos)
            + jnp.sum(neg * neg) + jnp.sum(ii * ii))

    n_pos = src_ref.shape[1] * pl.num_programs(0)
    n_neg = neg_ref.shape[0] * n_pos
    contrib = (s_pos / n_pos
               + (_NEG_WEIGHT / n_neg) * s_neg
               - _GAMMA * s_i
               + (0.5 * _L2_REG_WEIGHT) * s_l2)

    @pl.when(b == 0)
    def _():
        out_ref[0, 0] = 0.0
    out_ref[0, 0] += contrib


def _tc_loss(src_rows, pos_rows, neg_rows, ii_rows, ii_sc, bu, bip, bin_):
    b, d = src_rows.shape
    nneg = neg_rows.shape[0] // b
    topk = ii_rows.shape[0] // b
    bb = 128
    nblk = b // bb

    src3 = src_rows.reshape(1, b, d)
    pos3 = pos_rows.reshape(1, b, d)
    neg3 = neg_rows.reshape(nneg, b, d)
    ii3 = ii_rows.reshape(topk, b, d)
    bu2 = bu.reshape(1, b)
    bip2 = bip.reshape(1, b)
    bin2 = bin_.reshape(nneg, b)
    iisc2 = ii_sc.reshape(topk, b)

    out = pl.pallas_call(
        _loss_block,
        grid=(nblk,),
        in_specs=[
            pl.BlockSpec((1, bb, d), lambda i: (0, i, 0)),
            pl.BlockSpec((1, bb, d), lambda i: (0, i, 0)),
            pl.BlockSpec((nneg, bb, d), lambda i: (0, i, 0)),
            pl.BlockSpec((topk, bb, d), lambda i: (0, i, 0)),
            pl.BlockSpec((1, bb), lambda i: (0, i)),
            pl.BlockSpec((1, bb), lambda i: (0, i)),
            pl.BlockSpec((nneg, bb), lambda i: (0, i)),
            pl.BlockSpec((topk, bb), lambda i: (0, i)),
        ],
        out_specs=pl.BlockSpec((1, 1), lambda i: (0, 0),
                               memory_space=pltpu.SMEM),
        out_shape=jax.ShapeDtypeStruct((1, 1), jnp.float32),
    )(src3, pos3, neg3, ii3, bu2, bip2, bin2, iisc2)
    return out[0, 0]


def kernel(emb_table, beta_uD, beta_iD, ii_topk_similarity_scores,
           src, pos, neg, ii_topk_neighbors):
    b, nneg = neg.shape
    topk = ii_topk_neighbors.shape[1]

    neg_t = neg.T.reshape(-1).astype(jnp.int32)
    ii_flat_idx = (pos[None, :] * topk
                   + jnp.arange(topk, dtype=pos.dtype)[:, None]
                   ).reshape(-1).astype(jnp.int32)
    nbr_flat = ii_topk_neighbors.reshape(-1).astype(jnp.int32)
    sc_flat = ii_topk_similarity_scores.reshape(-1)

    (src_rows, pos_rows, neg_rows, ii_rows, ii_sc, bu, bip, bin_) = _sc_gather(
        emb_table, beta_uD, beta_iD, nbr_flat, sc_flat,
        src.astype(jnp.int32), pos.astype(jnp.int32), neg_t, ii_flat_idx)

    return _tc_loss(src_rows, pos_rows, neg_rows, ii_rows, ii_sc, bu, bip, bin_)


# speedup vs baseline: 6.4279x; 1.2779x over previous
import functools

import jax
import jax.numpy as jnp
from jax import lax
from jax.experimental import pallas as pl
from jax.experimental.pallas import tpu as pltpu
from jax.experimental.pallas import tpu_sc as plsc

_LAM = 0.75
_GAMMA = 1.5
_NEG_WEIGHT = 300.0
_L2_REG_WEIGHT = 1e-4

_NC = 2
_NS = 16
_NW = _NC * _NS
_CH = 128


def _sc_gather(emb_table, beta_uD, beta_iD, nbr_flat, sc_flat,
               src, pos, neg_t, ii_flat_idx):
    b = src.shape[0]
    nneg_total = neg_t.shape[0]
    nii_total = ii_flat_idx.shape[0]
    d = emb_table.shape[1]

    b_w = b // _NW
    neg_w = nneg_total // _NW
    ii_w = nii_total // _NW
    assert b_w == _CH and neg_w % _CH == 0 and ii_w % _CH == 0

    mesh = plsc.VectorSubcoreMesh(core_axis_name="c", subcore_axis_name="s")
    f32 = jnp.float32
    DMA = pltpu.SemaphoreType.DMA

    @functools.partial(
        pl.kernel,
        out_type=[
            jax.ShapeDtypeStruct((b, d), f32),
            jax.ShapeDtypeStruct((b, d), f32),
            jax.ShapeDtypeStruct((nneg_total, d), f32),
            jax.ShapeDtypeStruct((nii_total, d), f32),
            jax.ShapeDtypeStruct((nii_total,), f32),
            jax.ShapeDtypeStruct((b,), f32),
            jax.ShapeDtypeStruct((b,), f32),
            jax.ShapeDtypeStruct((nneg_total,), f32),
        ],
        mesh=mesh,
        scratch_types=[
            pltpu.VMEM((_CH,), jnp.int32),
            pltpu.VMEM((_CH,), jnp.int32),
            pltpu.VMEM((_CH, 128), f32),
            pltpu.VMEM((_CH, 128), f32),
            pltpu.VMEM((_CH,), f32),
            pltpu.VMEM((_CH,), f32),
            pltpu.VMEM((ii_w,), jnp.int32),
            DMA, DMA,
            DMA, DMA,
            DMA, DMA,
            DMA, DMA,
        ],
    )
    def gather_kernel(emb_h, bu_h, bi_h, nbr_h, scf_h, src_h, pos_h, negt_h,
                      iidx_h, osrc_h, opos_h, oneg_h, oii_h, oiisc_h,
                      obu_h, obip_h, obin_h,
                      idx0, idx1, rows0, rows1, val0, val1, nbr_v,
                      g0, g1, v0, v1, w0, w1, x0, x1):
        wid = lax.axis_index("s") * _NC + lax.axis_index("c")
        idx = (idx0, idx1)
        rows = (rows0, rows1)
        val = (val0, val1)
        gsem = (g0, g1)
        vsem = (v0, v1)
        wsem = (w0, w1)
        xsem = (x0, x1)

        def run_job(nchunks, start, finish):
            if nchunks == 2:
                start(0, 0, False)
                start(1, 1, False)
                finish(0, 0)
                finish(1, 1)
            else:
                start(0, 0, False)
                start(1, 1, False)

                @pl.loop(0, nchunks // 2)
                def _(g):
                    for s in range(2):
                        c = 2 * g + s
                        finish(s, c)

                        @pl.when(c + 2 < nchunks)
                        def _():
                            start(s, c + 2, True)

        def drain(sems_and_waits):
            for sem, src_ref, dst_ref in sems_and_waits:
                pltpu.make_async_copy(src_ref, dst_ref, sem).wait()

        abase = wid * b_w

        def a_start(s, c, wait_reuse):
            idx_h = src_h if c == 0 else pos_h
            beta_h = bu_h if c == 0 else bi_h
            pltpu.sync_copy(idx_h.at[pl.ds(abase, _CH)], idx[s])
            pltpu.make_async_copy(emb_h.at[idx[s]], rows[s], gsem[s]).start()
            pltpu.make_async_copy(beta_h.at[idx[s]], val[s], vsem[s]).start()

        def a_finish(s, c):
            orow_h = osrc_h if c == 0 else opos_h
            oval_h = obu_h if c == 0 else obip_h
            beta_h = bu_h if c == 0 else bi_h
            pltpu.make_async_copy(emb_h.at[idx[s]], rows[s], gsem[s]).wait()
            pltpu.make_async_copy(rows[s], orow_h.at[pl.ds(abase, _CH)],
                                  wsem[s]).start()
            pltpu.make_async_copy(beta_h.at[idx[s]], val[s], vsem[s]).wait()
            pltpu.make_async_copy(val[s], oval_h.at[pl.ds(abase, _CH)],
                                  xsem[s]).start()

        run_job(2, a_start, a_finish)
        drain([(wsem[0], rows[0], osrc_h.at[pl.ds(abase, _CH)]),
               (wsem[1], rows[1], opos_h.at[pl.ds(abase, _CH)]),
               (xsem[0], val[0], obu_h.at[pl.ds(abase, _CH)]),
               (xsem[1], val[1], obip_h.at[pl.ds(abase, _CH)])])

        def b_start(s, c, wait_reuse):
            nbase = wid * neg_w + c * _CH
            if wait_reuse:
                pltpu.make_async_copy(rows[s], oneg_h.at[pl.ds(0, _CH)],
                                      wsem[s]).wait()
                pltpu.make_async_copy(val[s], obin_h.at[pl.ds(0, _CH)],
                                      xsem[s]).wait()
            pltpu.sync_copy(negt_h.at[pl.ds(nbase, _CH)], idx[s])
            pltpu.make_async_copy(emb_h.at[idx[s]], rows[s], gsem[s]).start()
            pltpu.make_async_copy(bi_h.at[idx[s]], val[s], vsem[s]).start()

        def b_finish(s, c):
            nbase = wid * neg_w + c * _CH
            pltpu.make_async_copy(emb_h.at[idx[s]], rows[s], gsem[s]).wait()
            pltpu.make_async_copy(rows[s], oneg_h.at[pl.ds(nbase, _CH)],
                                  wsem[s]).start()
            pltpu.make_async_copy(bi_h.at[idx[s]], val[s], vsem[s]).wait()
            pltpu.make_async_copy(val[s], obin_h.at[pl.ds(nbase, _CH)],
                                  xsem[s]).start()

        run_job(neg_w // _CH, b_start, b_finish)
        drain([(wsem[0], rows[0], oneg_h.at[pl.ds(0, _CH)]),
               (wsem[1], rows[1], oneg_h.at[pl.ds(0, _CH)]),
               (xsem[0], val[0], obin_h.at[pl.ds(0, _CH)]),
               (xsem[1], val[1], obin_h.at[pl.ds(0, _CH)])])

        def c_start(s, c, wait_reuse):
            ibase = wid * ii_w + c * _CH
            if wait_reuse:
                pltpu.make_async_copy(val[s], oiisc_h.at[pl.ds(0, _CH)],
                                      xsem[s]).wait()
            pltpu.sync_copy(iidx_h.at[pl.ds(ibase, _CH)], idx[s])
            pltpu.make_async_copy(nbr_h.at[idx[s]],
                                  nbr_v.at[pl.ds(c * _CH, _CH)],
                                  gsem[s]).start()
            pltpu.make_async_copy(scf_h.at[idx[s]], val[s], vsem[s]).start()

        def c_finish(s, c):
            ibase = wid * ii_w + c * _CH
            pltpu.make_async_copy(nbr_h.at[idx[s]],
                                  nbr_v.at[pl.ds(c * _CH, _CH)],
                                  gsem[s]).wait()
            pltpu.make_async_copy(scf_h.at[idx[s]], val[s], vsem[s]).wait()
            pltpu.make_async_copy(val[s], oiisc_h.at[pl.ds(ibase, _CH)],
                                  xsem[s]).start()

        run_job(ii_w // _CH, c_start, c_finish)
        drain([(xsem[0], val[0], oiisc_h.at[pl.ds(0, _CH)]),
               (xsem[1], val[1], oiisc_h.at[pl.ds(0, _CH)])])

        def d_start(s, c, wait_reuse):
            if wait_reuse:
                pltpu.make_async_copy(rows[s], oii_h.at[pl.ds(0, _CH)],
                                      wsem[s]).wait()
            pltpu.make_async_copy(emb_h.at[nbr_v.at[pl.ds(c * _CH, _CH)]],
                                  rows[s], gsem[s]).start()

        def d_finish(s, c):
            ibase = wid * ii_w + c * _CH
            pltpu.make_async_copy(emb_h.at[nbr_v.at[pl.ds(c * _CH, _CH)]],
                                  rows[s], gsem[s]).wait()
            pltpu.make_async_copy(rows[s], oii_h.at[pl.ds(ibase, _CH)],
                                  wsem[s]).start()

        run_job(ii_w // _CH, d_start, d_finish)
        drain([(wsem[0], rows[0], oii_h.at[pl.ds(0, _CH)]),
               (wsem[1], rows[1], oii_h.at[pl.ds(0, _CH)])])

    return gather_kernel(emb_table, beta_uD, beta_iD, nbr_flat, sc_flat,
                         src, pos, neg_t, ii_flat_idx)


def _loss_block(src_ref, pos_ref, neg_ref, ii_ref, bu_ref, bip_ref,
                bin_ref, iisc_ref, out_ref):
    b = pl.program_id(0)

    src = src_ref[...]
    pos = pos_ref[...]
    neg = neg_ref[...]
    ii = ii_ref[...]

    pos_score = jnp.sum(src * pos, axis=-1)
    neg_score = jnp.sum(src * neg, axis=-1)
    ii_score = jnp.sum(src * ii, axis=-1)

    bu = bu_ref[...]
    bip = bip_ref[...]
    bin_ = bin_ref[...]
    iisc = iisc_ref[...]

    def bce(x, target):
        return (jnp.maximum(x, 0.0) - x * target
                + jnp.log1p(jnp.exp(-jnp.abs(x))))

    pos_coe = 1.0 + _LAM * bu * bip
    neg_coe = 1.0 + _LAM * bu * bin_
    s_pos = jnp.sum(bce(pos_score, 1.0) * pos_coe)
    s_neg = jnp.sum(bce(neg_score, 0.0) * neg_coe)

    log_sig = jnp.minimum(ii_score, 0.0) - jnp.log1p(jnp.exp(-jnp.abs(ii_score)))
    s_i = jnp.sum(iisc * log_sig)

    s_l2 = (jnp.sum(src * src) + jnp.sum(pos * pos)
            + jnp.sum(neg * neg) + jnp.sum(ii * ii))

    n_pos = src_ref.shape[1] * pl.num_programs(0)
    n_neg = neg_ref.shape[0] * n_pos
    contrib = (s_pos / n_pos
               + (_NEG_WEIGHT / n_neg) * s_neg
               - _GAMMA * s_i
               + (0.5 * _L2_REG_WEIGHT) * s_l2)

    @pl.when(b == 0)
    def _():
        out_ref[0, 0] = 0.0
    out_ref[0, 0] += contrib


def _tc_loss(src_rows, pos_rows, neg_rows, ii_rows, ii_sc, bu, bip, bin_):
    b, d = src_rows.shape
    nneg = neg_rows.shape[0] // b
    topk = ii_rows.shape[0] // b
    bb = 128
    nblk = b // bb

    src3 = src_rows.reshape(1, b, d)
    pos3 = pos_rows.reshape(1, b, d)
    neg3 = neg_rows.reshape(nneg, b, d)
    ii3 = ii_rows.reshape(topk, b, d)
    bu2 = bu.reshape(1, b)
    bip2 = bip.reshape(1, b)
    bin2 = bin_.reshape(nneg, b)
    iisc2 = ii_sc.reshape(topk, b)

    out = pl.pallas_call(
        _loss_block,
        grid=(nblk,),
        in_specs=[
            pl.BlockSpec((1, bb, d), lambda i: (0, i, 0)),
            pl.BlockSpec((1, bb, d), lambda i: (0, i, 0)),
            pl.BlockSpec((nneg, bb, d), lambda i: (0, i, 0)),
            pl.BlockSpec((topk, bb, d), lambda i: (0, i, 0)),
            pl.BlockSpec((1, bb), lambda i: (0, i)),
            pl.BlockSpec((1, bb), lambda i: (0, i)),
            pl.BlockSpec((nneg, bb), lambda i: (0, i)),
            pl.BlockSpec((topk, bb), lambda i: (0, i)),
        ],
        out_specs=pl.BlockSpec((1, 1), lambda i: (0, 0),
                               memory_space=pltpu.SMEM),
        out_shape=jax.ShapeDtypeStruct((1, 1), jnp.float32),
    )(src3, pos3, neg3, ii3, bu2, bip2, bin2, iisc2)
    return out[0, 0]


def kernel(emb_table, beta_uD, beta_iD, ii_topk_similarity_scores,
           src, pos, neg, ii_topk_neighbors):
    b, nneg = neg.shape
    topk = ii_topk_neighbors.shape[1]

    neg_t = neg.T.reshape(-1).astype(jnp.int32)
    ii_flat_idx = (pos[None, :] * topk
                   + jnp.arange(topk, dtype=pos.dtype)[:, None]
                   ).reshape(-1).astype(jnp.int32)
    nbr_flat = ii_topk_neighbors.reshape(-1).astype(jnp.int32)
    sc_flat = ii_topk_similarity_scores.reshape(-1)

    (src_rows, pos_rows, neg_rows, ii_rows, ii_sc, bu, bip, bin_) = _sc_gather(
        emb_table, beta_uD, beta_iD, nbr_flat, sc_flat,
        src.astype(jnp.int32), pos.astype(jnp.int32), neg_t, ii_flat_idx)

    return _tc_loss(src_rows, pos_rows, neg_rows, ii_rows, ii_sc, bu, bip, bin_)
